# baseline (speedup 1.0000x reference)
import functools

import jax
import jax.numpy as jnp
from jax import lax
from jax.experimental import pallas as pl
from jax.experimental.pallas import tpu as pltpu
from jax.experimental.pallas import tpu_sc as plsc

N = 10000
E = 160000
D = 128
NT = 5
R = 4
CHUNK = 128
NCH = E // CHUNK
NW = 32
CPT = ((-(-NCH // NW)) + 7) // 8 * 8
NCH_PAD = NW * CPT
ROWB = 640
ROWL = N - 15 * ROWB
NP = 16 * ROWB


def _mesh():
    return plsc.VectorSubcoreMesh(core_axis_name="c", subcore_axis_name="s")


def _for_rows(s, emit):
    @pl.when(s < 15)
    def _main():
        emit(s * ROWB, ROWB)

    @pl.when(s == 15)
    def _last():
        emit(15 * ROWB, ROWL)


def _deg_body(src_hbm, dst_hbm, ones_hbm, out_hbm, idx_s, idx_d,
              ones_v, zero_v, acc_o, acc_i):
    c = lax.axis_index("c")
    s = lax.axis_index("s")
    w = s * 2 + c

    pltpu.sync_copy(ones_hbm, ones_v)
    for k in range(ROWB // 16):
        zero_v[pl.ds(k * 16, 16)] = jnp.zeros((16,), jnp.float32)

    r0 = s * ROWB
    pltpu.sync_copy(zero_v, acc_o.at[pl.ds(r0, ROWB)])
    pltpu.sync_copy(zero_v, acc_i.at[pl.ds(r0, ROWB)])
    plsc.subcore_barrier()

    c0 = w * CPT
    cnt = jnp.clip(NCH - c0, 0, CPT)
    pltpu.sync_copy(src_hbm.at[pl.ds(c0, CPT)], idx_s)
    pltpu.sync_copy(dst_hbm.at[pl.ds(c0, CPT)], idx_d)

    def body(j, carry):
        @pl.when(j < cnt)
        def _go():
            pltpu.sync_copy(ones_v, acc_o.at[idx_s.at[j]], add=True)
            pltpu.sync_copy(ones_v, acc_i.at[idx_d.at[j]], add=True)
        return carry
    lax.fori_loop(0, CPT, body, None)
    plsc.subcore_barrier()

    pltpu.sync_copy(acc_o.at[pl.ds(r0, ROWB)],
                    out_hbm.at[c, 0, pl.ds(r0, ROWB)])
    pltpu.sync_copy(acc_i.at[pl.ds(r0, ROWB)],
                    out_hbm.at[c, 1, pl.ds(r0, ROWB)])


_deg_kernel = pl.kernel(
    _deg_body,
    out_type=jax.ShapeDtypeStruct((2, 2, NP), jnp.float32),
    mesh=_mesh(),
    scratch_types=[
        pltpu.VMEM((CPT, CHUNK), jnp.int32),
        pltpu.VMEM((CPT, CHUNK), jnp.int32),
        pltpu.VMEM((CHUNK,), jnp.float32),
        pltpu.VMEM((ROWB,), jnp.float32),
        pltpu.VMEM_SHARED((NP,), jnp.float32),
        pltpu.VMEM_SHARED((NP,), jnp.float32),
    ],
)


NB = 2
LA = NB // 2
NE = N + 128


def _edge_pass_body(table_hbm, src_hbm, dst_hbm, zeros_hbm, out_hbm,
                    idx_s, idx_d, rows_v, acc, *sems):
    sem_g = sems[:NB]
    sem_s = sems[NB:]
    c = lax.axis_index("c")
    s = lax.axis_index("s")
    w = s * 2 + c

    def zero(r0, nr):
        pltpu.sync_copy(zeros_hbm.at[pl.ds(r0, nr)], acc.at[pl.ds(r0, nr)])
    _for_rows(s, zero)
    plsc.subcore_barrier()

    c0 = w * CPT
    pltpu.sync_copy(src_hbm.at[pl.ds(c0, CPT)], idx_s)
    pltpu.sync_copy(dst_hbm.at[pl.ds(c0, CPT)], idx_d)

    def g_desc(j, b):
        return pltpu.make_async_copy(
            table_hbm.at[idx_s.at[j]], rows_v.at[b], sem_g[b])

    def s_desc(j, b):
        return pltpu.make_async_copy(
            rows_v.at[b], acc.at[idx_d.at[j]], sem_s[b])

    for b in range(LA):
        g_desc(b, b).start()

    def group(g, carry):
        for b in range(NB):
            j = g * NB + b
            pb = (b + LA) % NB
            g_desc(j, b).wait()
            pltpu.async_copy(rows_v.at[b], acc.at[idx_d.at[j]],
                             sem_s[b], add=True)

            @pl.when(j >= LA)
            def _drain():
                s_desc(j - LA, pb).wait()

            @pl.when(j + LA < CPT)
            def _prefetch():
                g_desc(j + LA, pb).start()
        return carry
    lax.fori_loop(0, CPT // NB, group, None)
    for j in range(CPT - LA, CPT):
        s_desc(j, j % NB).wait()
    plsc.subcore_barrier()

    def wr(r0, nr):
        pltpu.sync_copy(acc.at[pl.ds(r0, nr)], out_hbm.at[c, pl.ds(r0, nr)])
    _for_rows(s, wr)


def _make_edge_pass():
    scratch = [
        pltpu.VMEM((CPT, CHUNK), jnp.int32),
        pltpu.VMEM((CPT, CHUNK), jnp.int32),
        pltpu.VMEM((NB, CHUNK, D), jnp.float32),
        pltpu.VMEM_SHARED((NE, D), jnp.float32),
    ]
    scratch += [pltpu.SemaphoreType.DMA] * (2 * NB)
    return pl.kernel(
        _edge_pass_body,
        out_type=jax.ShapeDtypeStruct((2, N, D), jnp.float32),
        mesh=_mesh(),
        scratch_types=scratch,
    )


_edge_pass = _make_edge_pass()


def _deg_combine_body(dp_ref, o_ref):
    x = dp_ref[...]
    o_ref[...] = x[0] + x[1]


def _deg_combine(dp):
    return pl.pallas_call(
        _deg_combine_body,
        out_shape=jax.ShapeDtypeStruct((2, NP), jnp.float32),
    )(dp)


def _proj_body(x_ref, w_ref, b_ref, deg_ref, o_ref):
    a = lax.rsqrt(jnp.maximum(deg_ref[...], 1.0))
    y = jnp.dot(x_ref[...], w_ref[...], preferred_element_type=jnp.float32)
    o_ref[...] = (y + b_ref[...]) * a


def _tc_proj(x, W, b, deg):
    return pl.pallas_call(
        _proj_body,
        out_shape=jax.ShapeDtypeStruct((x.shape[0], D), jnp.float32),
    )(x, W, b.reshape(1, D), deg)


def _rel_body(pa_ref, pb_ref, dego_ref, degi_ref, coef_ref, basis_ref, o_ref):
    a = lax.rsqrt(jnp.maximum(dego_ref[...], 1.0))
    b = lax.rsqrt(jnp.maximum(degi_ref[...], 1.0))
    z = (pa_ref[...] + pb_ref[...]) * (a * b)
    bas = basis_ref[...].reshape(R, D * D)
    cw = coef_ref[pl.ds(pl.program_id(0), 1), :]
    wt = jnp.dot(cw, bas,
                 preferred_element_type=jnp.float32).reshape(D, D)
    o_ref[...] = jnp.dot(z, wt, preferred_element_type=jnp.float32)[None]


def _tc_rel(pa, pb, dego, degi, coef, basis):
    return pl.pallas_call(
        _rel_body,
        grid=(NT,),
        in_specs=[
            pl.BlockSpec((N, D), lambda t: (0, 0)),
            pl.BlockSpec((N, D), lambda t: (0, 0)),
            pl.BlockSpec((N, 1), lambda t: (0, 0)),
            pl.BlockSpec((N, 1), lambda t: (0, 0)),
            pl.BlockSpec((NT, R), lambda t: (0, 0)),
            pl.BlockSpec((R, D, D), lambda t: (0, 0, 0)),
        ],
        out_specs=pl.BlockSpec((1, N, D), lambda t: (t, 0, 0)),
        out_shape=jax.ShapeDtypeStruct((NT, N, D), jnp.float32),
    )(pa, pb, dego, degi, coef, basis)


def _relu_body(qa_ref, qb_ref, dego_ref, degi_ref, bias_ref, o_ref):
    a = lax.rsqrt(jnp.maximum(dego_ref[...], 1.0))
    b = lax.rsqrt(jnp.maximum(degi_ref[...], 1.0))
    h2 = jnp.maximum(b * (qa_ref[...] + qb_ref[...]) + bias_ref[...], 0.0)
    o_ref[...] = a * h2


def _tc_relu(qa, qb, dego, degi, bias1):
    return pl.pallas_call(
        _relu_body,
        out_shape=jax.ShapeDtypeStruct((N, D), jnp.float32),
    )(qa, qb, dego, degi, bias1.reshape(1, D))


def _out_body(ra_ref, rb_ref, degi_ref, wout_ref, bout_ref, o1_ref, o2_ref):
    b = lax.rsqrt(jnp.maximum(degi_ref[...], 1.0))
    h3 = (ra_ref[...] + rb_ref[...]) * b
    o2_ref[...] = h3
    o1_ref[...] = jnp.dot(h3, wout_ref[...],
                          preferred_element_type=jnp.float32) + bout_ref[...]


def _tc_out(ra, rb, degi, Wout, bout):
    n_classes = Wout.shape[1]
    return pl.pallas_call(
        _out_body,
        out_shape=[
            jax.ShapeDtypeStruct((N, n_classes), jnp.float32),
            jax.ShapeDtypeStruct((N, D), jnp.float32),
        ],
    )(ra, rb, degi, Wout, bout.reshape(1, n_classes))


def _pad_idx(x, base):
    pad = NCH_PAD * CHUNK - E
    padv = base + (jnp.arange(pad, dtype=jnp.int32) % 128)
    return jnp.concatenate([x, padv]).reshape(NCH_PAD, CHUNK)


def kernel(feat0, feat1, feat2, W0, b0, W1, b1, W2, b2, coef, basis, bias1,
           Wout, bout, src, dst, e_feat):
    srcp = _pad_idx(src, 0)
    dstp = _pad_idx(dst, N)
    src2p = _pad_idx(e_feat * N + src, 0)
    zerosD = jnp.zeros((N, D), jnp.float32)

    ones1 = jnp.ones((CHUNK,), jnp.float32)
    dp = _deg_kernel(srcp, dstp, ones1)
    degs = _deg_combine(dp)
    dego = degs[0, :N].reshape(N, 1)
    degi = degs[1, :N].reshape(N, 1)

    n0 = feat0.shape[0]
    n1 = feat1.shape[0]
    g0 = jnp.concatenate([
        _tc_proj(feat0, W0, b0, dego[:n0]),
        _tc_proj(feat1, W1, b1, dego[n0:n0 + n1]),
        _tc_proj(feat2, W2, b2, dego[n0 + n1:]),
    ], axis=0)

    p = _edge_pass(g0, srcp, dstp, zerosD)
    G = _tc_rel(p[0], p[1], dego, degi, coef, basis)
    q = _edge_pass(G.reshape(NT * N, D), src2p, dstp, zerosD)
    g2 = _tc_relu(q[0], q[1], dego, degi, bias1)
    r = _edge_pass(g2, srcp, dstp, zerosD)
    logits, h3 = _tc_out(r[0], r[1], degi, Wout, bout)
    return (logits, h3)


# speedup vs baseline: 1.1157x; 1.1157x over previous
import functools

import jax
import jax.numpy as jnp
from jax import lax
from jax.experimental import pallas as pl
from jax.experimental.pallas import tpu as pltpu
from jax.experimental.pallas import tpu_sc as plsc

N = 10000
E = 160000
D = 128
NT = 5
R = 4
CHUNK = 128
NCH = E // CHUNK
NW = 32
CPT = ((-(-NCH // NW)) + 7) // 8 * 8
NCH_PAD = NW * CPT
ROWB = 640
ROWL = N - 15 * ROWB
NP = 16 * ROWB


def _mesh():
    return plsc.VectorSubcoreMesh(core_axis_name="c", subcore_axis_name="s")


def _for_rows(s, emit):
    @pl.when(s < 15)
    def _main():
        emit(s * ROWB, ROWB)

    @pl.when(s == 15)
    def _last():
        emit(15 * ROWB, ROWL)


def _deg_body(src_hbm, dst_hbm, ones_hbm, out_hbm, idx_s, idx_d,
              ones_v, zero_v, acc_o, acc_i):
    c = lax.axis_index("c")
    s = lax.axis_index("s")
    w = s * 2 + c

    pltpu.sync_copy(ones_hbm, ones_v)
    for k in range(ROWB // 16):
        zero_v[pl.ds(k * 16, 16)] = jnp.zeros((16,), jnp.float32)

    r0 = s * ROWB
    pltpu.sync_copy(zero_v, acc_o.at[pl.ds(r0, ROWB)])
    pltpu.sync_copy(zero_v, acc_i.at[pl.ds(r0, ROWB)])
    plsc.subcore_barrier()

    c0 = w * CPT
    cnt = jnp.clip(NCH - c0, 0, CPT)
    pltpu.sync_copy(src_hbm.at[pl.ds(c0, CPT)], idx_s)
    pltpu.sync_copy(dst_hbm.at[pl.ds(c0, CPT)], idx_d)

    def body(j, carry):
        @pl.when(j < cnt)
        def _go():
            pltpu.sync_copy(ones_v, acc_o.at[idx_s.at[j]], add=True)
            pltpu.sync_copy(ones_v, acc_i.at[idx_d.at[j]], add=True)
        return carry
    lax.fori_loop(0, CPT, body, None)
    plsc.subcore_barrier()

    pltpu.sync_copy(acc_o.at[pl.ds(r0, ROWB)],
                    out_hbm.at[c, 0, pl.ds(r0, ROWB)])
    pltpu.sync_copy(acc_i.at[pl.ds(r0, ROWB)],
                    out_hbm.at[c, 1, pl.ds(r0, ROWB)])


_deg_kernel = pl.kernel(
    _deg_body,
    out_type=jax.ShapeDtypeStruct((2, 2, NP), jnp.float32),
    mesh=_mesh(),
    scratch_types=[
        pltpu.VMEM((CPT, CHUNK), jnp.int32),
        pltpu.VMEM((CPT, CHUNK), jnp.int32),
        pltpu.VMEM((CHUNK,), jnp.float32),
        pltpu.VMEM((ROWB,), jnp.float32),
        pltpu.VMEM_SHARED((NP,), jnp.float32),
        pltpu.VMEM_SHARED((NP,), jnp.float32),
    ],
)


NB = 2
LA = NB // 2
NE = N + 128


def _edge_pass_body(table_hbm, src_hbm, dst_hbm, zeros_hbm, out_hbm,
                    idx_s, idx_d, rows_v, acc, *sems):
    sem_g = sems[:NB]
    sem_s = sems[NB:]
    c = lax.axis_index("c")
    s = lax.axis_index("s")
    w = s * 2 + c

    def zero(r0, nr):
        pltpu.sync_copy(zeros_hbm.at[pl.ds(r0, nr)], acc.at[pl.ds(r0, nr)])
    _for_rows(s, zero)
    plsc.subcore_barrier()

    c0 = w * CPT
    pltpu.sync_copy(src_hbm.at[pl.ds(c0, CPT)], idx_s)
    pltpu.sync_copy(dst_hbm.at[pl.ds(c0, CPT)], idx_d)

    def g_desc(j, b):
        return pltpu.make_async_copy(
            table_hbm.at[idx_s.at[j]], rows_v.at[b], sem_g[b])

    def s_desc(j, b):
        return pltpu.make_async_copy(
            rows_v.at[b], acc.at[idx_d.at[j]], sem_s[b])

    for b in range(LA):
        g_desc(b, b).start()

    def group(g, carry):
        for b in range(NB):
            j = g * NB + b
            pb = (b + LA) % NB
            g_desc(j, b).wait()
            pltpu.async_copy(rows_v.at[b], acc.at[idx_d.at[j]],
                             sem_s[b], add=True)

            @pl.when(j >= LA)
            def _drain():
                s_desc(j - LA, pb).wait()

            @pl.when(j + LA < CPT)
            def _prefetch():
                g_desc(j + LA, pb).start()
        return carry
    lax.fori_loop(0, CPT // NB, group, None)
    for j in range(CPT - LA, CPT):
        s_desc(j, j % NB).wait()
    plsc.subcore_barrier()

    def wr(r0, nr):
        pltpu.sync_copy(acc.at[pl.ds(r0, nr)], out_hbm.at[c, pl.ds(r0, nr)])
    _for_rows(s, wr)


def _make_edge_pass():
    scratch = [
        pltpu.VMEM((CPT, CHUNK), jnp.int32),
        pltpu.VMEM((CPT, CHUNK), jnp.int32),
        pltpu.VMEM((NB, CHUNK, D), jnp.float32),
        pltpu.VMEM_SHARED((NE, D), jnp.float32),
    ]
    scratch += [pltpu.SemaphoreType.DMA] * (2 * NB)
    return pl.kernel(
        _edge_pass_body,
        out_type=jax.ShapeDtypeStruct((2, N, D), jnp.float32),
        mesh=_mesh(),
        scratch_types=scratch,
    )


_edge_pass = _make_edge_pass()


def _deg_combine_body(dp_ref, o_ref):
    x = dp_ref[...]
    o_ref[...] = x[0] + x[1]


def _deg_combine(dp):
    return pl.pallas_call(
        _deg_combine_body,
        out_shape=jax.ShapeDtypeStruct((2, NP), jnp.float32),
    )(dp)


def _proj_body(x_ref, w_ref, b_ref, o_ref):
    y = jnp.dot(x_ref[...], w_ref[...], preferred_element_type=jnp.float32)
    o_ref[...] = y + b_ref[...]


def _tc_proj(x, W, b):
    return pl.pallas_call(
        _proj_body,
        out_shape=jax.ShapeDtypeStruct((x.shape[0], D), jnp.float32),
    )(x, W, b.reshape(1, D))


def _gcat_body(p0_ref, p1_ref, p2_ref, dego_ref, o_ref):
    a = lax.rsqrt(jnp.maximum(dego_ref[...], 1.0))
    n0 = p0_ref.shape[0]
    n1 = p1_ref.shape[0]
    o_ref[0:n0, :] = p0_ref[...] * a[0:n0]
    o_ref[n0:n0 + n1, :] = p1_ref[...] * a[n0:n0 + n1]
    o_ref[n0 + n1:N, :] = p2_ref[...] * a[n0 + n1:N]


def _tc_gcat(p0, p1, p2, dego):
    return pl.pallas_call(
        _gcat_body,
        out_shape=jax.ShapeDtypeStruct((N, D), jnp.float32),
    )(p0, p1, p2, dego)


def _rel_body(p_ref, dego_ref, degi_ref, coef_ref, basis_ref, o_ref, z_ref):
    t = pl.program_id(0)

    @pl.when(t == 0)
    def _mkz():
        a = lax.rsqrt(jnp.maximum(dego_ref[...], 1.0))
        b = lax.rsqrt(jnp.maximum(degi_ref[...], 1.0))
        z_ref[...] = (p_ref[0] + p_ref[1]) * (a * b)

    bas = basis_ref[...].reshape(R, D * D)
    cw = coef_ref[pl.ds(t, 1), :]
    wt = jnp.dot(cw, bas,
                 preferred_element_type=jnp.float32).reshape(D, D)
    o_ref[...] = jnp.dot(z_ref[...], wt, preferred_element_type=jnp.float32)


def _tc_rel(p, dego, degi, coef, basis):
    return pl.pallas_call(
        _rel_body,
        grid=(NT,),
        in_specs=[
            pl.BlockSpec((2, N, D), lambda t: (0, 0, 0)),
            pl.BlockSpec((N, 1), lambda t: (0, 0)),
            pl.BlockSpec((N, 1), lambda t: (0, 0)),
            pl.BlockSpec((NT, R), lambda t: (0, 0)),
            pl.BlockSpec((R, D, D), lambda t: (0, 0, 0)),
        ],
        out_specs=pl.BlockSpec((N, D), lambda t: (t, 0)),
        out_shape=jax.ShapeDtypeStruct((NT * N, D), jnp.float32),
        scratch_shapes=[pltpu.VMEM((N, D), jnp.float32)],
    )(p, dego, degi, coef, basis)


def _relu_body(q_ref, dego_ref, degi_ref, bias_ref, o_ref):
    a = lax.rsqrt(jnp.maximum(dego_ref[...], 1.0))
    b = lax.rsqrt(jnp.maximum(degi_ref[...], 1.0))
    h2 = jnp.maximum(b * (q_ref[0] + q_ref[1]) + bias_ref[...], 0.0)
    o_ref[...] = a * h2


def _tc_relu(q, dego, degi, bias1):
    return pl.pallas_call(
        _relu_body,
        out_shape=jax.ShapeDtypeStruct((N, D), jnp.float32),
    )(q, dego, degi, bias1.reshape(1, D))


def _out_body(r_ref, degi_ref, wout_ref, bout_ref, o1_ref, o2_ref):
    b = lax.rsqrt(jnp.maximum(degi_ref[...], 1.0))
    h3 = (r_ref[0] + r_ref[1]) * b
    o2_ref[...] = h3
    o1_ref[...] = jnp.dot(h3, wout_ref[...],
                          preferred_element_type=jnp.float32) + bout_ref[...]


def _tc_out(r, degi, Wout, bout):
    n_classes = Wout.shape[1]
    return pl.pallas_call(
        _out_body,
        out_shape=[
            jax.ShapeDtypeStruct((N, n_classes), jnp.float32),
            jax.ShapeDtypeStruct((N, D), jnp.float32),
        ],
    )(r, degi, Wout, bout.reshape(1, n_classes))


def _pad_idx(x, base):
    pad = NCH_PAD * CHUNK - E
    padv = base + (jnp.arange(pad, dtype=jnp.int32) % 128)
    return jnp.concatenate([x, padv]).reshape(NCH_PAD, CHUNK)


def kernel(feat0, feat1, feat2, W0, b0, W1, b1, W2, b2, coef, basis, bias1,
           Wout, bout, src, dst, e_feat):
    srcp = _pad_idx(src, 0)
    dstp = _pad_idx(dst, N)
    src2p = _pad_idx(e_feat * N + src, 0)
    zerosD = jnp.zeros((N, D), jnp.float32)

    ones1 = jnp.ones((CHUNK,), jnp.float32)
    dp = _deg_kernel(srcp, dstp, ones1)
    h0a = _tc_proj(feat0, W0, b0)
    h0b = _tc_proj(feat1, W1, b1)
    h0c = _tc_proj(feat2, W2, b2)
    degs = _deg_combine(dp)
    dego = degs[0, :N].reshape(N, 1)
    degi = degs[1, :N].reshape(N, 1)

    g0 = _tc_gcat(h0a, h0b, h0c, dego)

    p = _edge_pass(g0, srcp, dstp, zerosD)
    G = _tc_rel(p, dego, degi, coef, basis)
    q = _edge_pass(G, src2p, dstp, zerosD)
    g2 = _tc_relu(q, dego, degi, bias1)
    r = _edge_pass(g2, srcp, dstp, zerosD)
    logits, h3 = _tc_out(r, degi, Wout, bout)
    return (logits, h3)
